# baseline (speedup 1.0000x reference)
import functools

import jax
import jax.numpy as jnp
from jax import lax
from jax.experimental import pallas as pl
from jax.experimental.pallas import tpu as pltpu
from jax.experimental.pallas import tpu_sc as plsc

B_ROWS = 4096
NUM_FIELDS = 26
EMBED_DIM = 64
PAD_DIM = 128
FIELD_SIZE = 40000
TABLE_ROWS = NUM_FIELDS * FIELD_SIZE
B = B_ROWS * NUM_FIELDS

NC = 2
NS = 16
NW = NC * NS
B_PER_W = B // NW
CHUNK = 832
N_CHUNKS = B_PER_W // CHUNK
LANES = 16
VECS_PER_CHUNK = CHUNK // LANES

TR_COLS = 3200
TR_GRID = TABLE_ROWS // TR_COLS


def _tr_body(t_ref, o_ref):
    blk = t_ref[...]
    pad = jnp.concatenate(
        [blk, jnp.zeros((EMBED_DIM, TR_COLS), jnp.float32)], axis=0
    )
    o_ref[...] = pad.T


def _transpose_pad(tt):
    return pl.pallas_call(
        _tr_body,
        grid=(TR_GRID,),
        in_specs=[pl.BlockSpec((EMBED_DIM, TR_COLS), lambda i: (0, i))],
        out_specs=pl.BlockSpec((TR_COLS, PAD_DIM), lambda i: (i, 0)),
        out_shape=jax.ShapeDtypeStruct((TABLE_ROWS, PAD_DIM), jnp.float32),
        compiler_params=pltpu.CompilerParams(
            dimension_semantics=("arbitrary",),
        ),
    )(tt)


def _gather_body(x_hbm, table_hbm, out_hbm, xv, idxv, rowsv, sem):
    wid = lax.axis_index("s") * NC + lax.axis_index("c")
    lane = lax.iota(jnp.int32, LANES)

    def do_chunk(c, _):
        base = wid * B_PER_W + c * CHUNK
        pltpu.sync_copy(x_hbm.at[pl.ds(base, CHUNK)], xv)

        def add_offsets(j, _):
            pos = base + j * LANES + lane
            field = lax.rem(pos, NUM_FIELDS)
            idxv[pl.ds(j * LANES, LANES)] = (
                xv[pl.ds(j * LANES, LANES)] + field * FIELD_SIZE
            )
            return 0

        lax.fori_loop(0, VECS_PER_CHUNK, add_offsets, 0)
        pltpu.async_copy(table_hbm.at[idxv], rowsv, sem).wait()
        pltpu.sync_copy(rowsv, out_hbm.at[pl.ds(base, CHUNK)])
        return 0

    lax.fori_loop(0, N_CHUNKS, do_chunk, 0)


def _sc_gather(x_flat, tp):
    mesh = plsc.VectorSubcoreMesh(core_axis_name="c", subcore_axis_name="s")
    k = functools.partial(
        pl.kernel,
        mesh=mesh,
        out_type=jax.ShapeDtypeStruct((B, PAD_DIM), jnp.float32),
        scratch_types=[
            pltpu.VMEM((CHUNK,), jnp.int32),
            pltpu.VMEM((CHUNK,), jnp.int32),
            pltpu.VMEM((CHUNK, PAD_DIM), jnp.float32),
            pltpu.SemaphoreType.DMA,
        ],
        compiler_params=pltpu.CompilerParams(use_tc_tiling_on_sc=False),
    )(_gather_body)
    return k(x_flat, tp)


@jax.jit
def kernel(x, table):
    tt = table.T
    tp = _transpose_pad(tt)
    out128 = _sc_gather(x.reshape(-1), tp)
    return out128[:, :EMBED_DIM].reshape(B_ROWS, NUM_FIELDS, EMBED_DIM)


# speedup vs baseline: 1.6963x; 1.6963x over previous
import functools

import jax
import jax.numpy as jnp
from jax import lax
from jax.experimental import pallas as pl
from jax.experimental.pallas import tpu as pltpu
from jax.experimental.pallas import tpu_sc as plsc

B_ROWS = 4096
NUM_FIELDS = 26
EMBED_DIM = 64
FIELD_SIZE = 40000
B = B_ROWS * NUM_FIELDS

NC = 2
NS = 16
NW = NC * NS
BB = B_ROWS // NW
NFPC = 2
N_CHUNKS = NUM_FIELDS // NFPC
CHUNK = NFPC * BB
LANES = 16
VECS_PER_BLK = BB // LANES

DBLK_STRIDE = 8320000
CB_STRIDE = 1024
DIN_STRIDE = 128


def _body(x_hbm, tbl_hbm, out_hbm, xv, idx0, idx1, rows0, rows1, sem0, sem1):
    wid = lax.axis_index("s") * NC + lax.axis_index("c")
    lane = lax.iota(jnp.int32, LANES)
    bbase = wid * BB

    def stage(c, idxv):
        pltpu.sync_copy(
            x_hbm.at[pl.ds(NFPC * c * B_ROWS + bbase, BB)],
            xv.at[pl.ds(0, BB)],
        )
        pltpu.sync_copy(
            x_hbm.at[pl.ds((NFPC * c + 1) * B_ROWS + bbase, BB)],
            xv.at[pl.ds(BB, BB)],
        )

        def mk_base(m, _):
            field = NFPC * c + m // VECS_PER_BLK
            row = xv[pl.ds(m * LANES, LANES)] + field * FIELD_SIZE
            cb = lax.shift_right_logical(row, 7)
            jl = lax.bitwise_and(row, 127)
            xv[pl.ds(m * LANES, LANES)] = cb * CB_STRIDE + jl
            return 0

        lax.fori_loop(0, NFPC * VECS_PER_BLK, mk_base, 0)

        def mk_idx(d, _):
            p = (d // 8) * DBLK_STRIDE + (d % 8) * DIN_STRIDE

            def mk_idx_vec(m, _):
                idxv[d, pl.ds(m * LANES, LANES)] = (
                    xv[pl.ds(m * LANES, LANES)] + p
                )
                return 0

            lax.fori_loop(0, NFPC * VECS_PER_BLK, mk_idx_vec, 0)
            return 0

        lax.fori_loop(0, EMBED_DIM, mk_idx, 0)

    def fire(idxv, rowsv, sem):
        def f(d, _):
            pltpu.async_copy(tbl_hbm.at[idxv.at[d]], rowsv.at[d], sem)
            return 0

        lax.fori_loop(0, EMBED_DIM, f, 0)

    def drain_write(c, rowsv, sem):
        pltpu.make_async_copy(
            out_hbm.at[0, :, pl.ds(0, CHUNK)], rowsv, sem
        ).wait()
        pltpu.sync_copy(
            rowsv.at[:, pl.ds(0, BB)],
            out_hbm.at[NFPC * c, :, pl.ds(bbase, BB)],
        )
        pltpu.sync_copy(
            rowsv.at[:, pl.ds(BB, BB)],
            out_hbm.at[NFPC * c + 1, :, pl.ds(bbase, BB)],
        )

    stage(0, idx0)
    fire(idx0, rows0, sem0)

    def pipeline(c, _):
        is_odd = lax.rem(c, 2) == 1

        @pl.when(is_odd)
        def _():
            stage(c, idx1)
            fire(idx1, rows1, sem1)
            drain_write(c - 1, rows0, sem0)

        @pl.when(jnp.logical_not(is_odd))
        def _():
            stage(c, idx0)
            fire(idx0, rows0, sem0)
            drain_write(c - 1, rows1, sem1)

        return 0

    lax.fori_loop(1, N_CHUNKS, pipeline, 0)
    drain_write(N_CHUNKS - 1, rows0, sem0)


@jax.jit
def kernel(x, table):
    tflat = (
        table.T.reshape(8, 8, 8125, 128).transpose(0, 2, 1, 3).reshape(-1)
    )
    mesh = plsc.VectorSubcoreMesh(core_axis_name="c", subcore_axis_name="s")
    k = functools.partial(
        pl.kernel,
        mesh=mesh,
        out_type=jax.ShapeDtypeStruct(
            (NUM_FIELDS, EMBED_DIM, B_ROWS), jnp.float32
        ),
        scratch_types=[
            pltpu.VMEM((CHUNK,), jnp.int32),
            pltpu.VMEM((EMBED_DIM, CHUNK), jnp.int32),
            pltpu.VMEM((EMBED_DIM, CHUNK), jnp.int32),
            pltpu.VMEM((EMBED_DIM, CHUNK), jnp.float32),
            pltpu.VMEM((EMBED_DIM, CHUNK), jnp.float32),
            pltpu.SemaphoreType.DMA,
            pltpu.SemaphoreType.DMA,
        ],
        compiler_params=pltpu.CompilerParams(use_tc_tiling_on_sc=False),
    )(_body)
    out3 = k(x.T.reshape(-1), tflat)
    return out3.transpose(2, 0, 1)


# speedup vs baseline: 1.8693x; 1.1020x over previous
import functools

import jax
import jax.numpy as jnp
from jax import lax
from jax.experimental import pallas as pl
from jax.experimental.pallas import tpu as pltpu
from jax.experimental.pallas import tpu_sc as plsc

B_ROWS = 4096
NUM_FIELDS = 26
EMBED_DIM = 64
FIELD_SIZE = 40000
B = B_ROWS * NUM_FIELDS

NC = 2
NS = 16
NW = NC * NS
BB = B_ROWS // NW
NFPC = 2
N_CHUNKS = NUM_FIELDS // NFPC
CHUNK = NFPC * BB
LANES = 16
VECS_PER_BLK = BB // LANES

DBLK_STRIDE = 8320000
CB_STRIDE = 1024
DIN_STRIDE = 128


def _body(
    x_hbm, tbl_hbm, tdum_hbm, out_hbm,
    xv, idx0, idx1, rows0, rows1, sem0, sem1, semw0, semw1,
):
    wid = lax.axis_index("s") * NC + lax.axis_index("c")
    lane = lax.iota(jnp.int32, LANES)
    bbase = wid * BB

    def stage(c, idxv):
        pltpu.sync_copy(
            x_hbm.at[pl.ds(NFPC * c * B_ROWS + bbase, BB)],
            xv.at[pl.ds(0, BB)],
        )
        pltpu.sync_copy(
            x_hbm.at[pl.ds((NFPC * c + 1) * B_ROWS + bbase, BB)],
            xv.at[pl.ds(BB, BB)],
        )

        def mk_base(m, _):
            field = NFPC * c + m // VECS_PER_BLK
            row = xv[pl.ds(m * LANES, LANES)] + field * FIELD_SIZE
            cb = lax.shift_right_logical(row, 7)
            jl = lax.bitwise_and(row, 127)
            xv[pl.ds(m * LANES, LANES)] = cb * CB_STRIDE + jl
            return 0

        lax.fori_loop(0, NFPC * VECS_PER_BLK, mk_base, 0)

        def mk_idx(d, _):
            p = (d // 8) * DBLK_STRIDE + (d % 8) * DIN_STRIDE

            def mk_idx_vec(m, _):
                idxv[d, pl.ds(m * LANES, LANES)] = (
                    xv[pl.ds(m * LANES, LANES)] + p
                )
                return 0

            lax.fori_loop(0, NFPC * VECS_PER_BLK, mk_idx_vec, 0)
            return 0

        lax.fori_loop(0, EMBED_DIM, mk_idx, 0)

    def fire(idxv, rowsv, sem):
        def f(d, _):
            pltpu.async_copy(
                tbl_hbm.at[idxv.at[d]], rowsv.at[d // 8, d % 8], sem
            )
            return 0

        lax.fori_loop(0, EMBED_DIM, f, 0)

    def drain(rowsv, sem):
        pltpu.make_async_copy(tdum_hbm.at[pl.ds(0, 8)], rowsv, sem).wait()

    def fire_writes(c, rowsv, semw):
        for fi in range(NFPC):
            f = NFPC * c + fi

            def w(dblk, _):
                pltpu.async_copy(
                    rowsv.at[dblk, :, pl.ds(fi * BB, BB)],
                    out_hbm.at[f, dblk, wid],
                    semw,
                )
                return 0

            lax.fori_loop(0, 8, w, 0)

    stage(0, idx0)
    fire(idx0, rows0, sem0)

    def pipeline(c, _):
        is_odd = lax.rem(c, 2) == 1

        @pl.when(is_odd)
        def _():
            @pl.when(c >= 2)
            def _():
                drain(rows1, semw1)
            stage(c, idx1)
            fire(idx1, rows1, sem1)
            drain(rows0, sem0)
            fire_writes(c - 1, rows0, semw0)

        @pl.when(jnp.logical_not(is_odd))
        def _():
            drain(rows0, semw0)
            stage(c, idx0)
            fire(idx0, rows0, sem0)
            drain(rows1, sem1)
            fire_writes(c - 1, rows1, semw1)

        return 0

    lax.fori_loop(1, N_CHUNKS, pipeline, 0)
    drain(rows0, sem0)
    fire_writes(N_CHUNKS - 1, rows0, semw0)
    drain(rows0, semw0)
    drain(rows1, semw1)


@jax.jit
def kernel(x, table):
    tflat = (
        table.T.reshape(8, 8, 8125, 128).transpose(0, 2, 1, 3).reshape(-1)
    )
    tdum = tflat[: 8 * 8 * CHUNK].reshape(8, 8, CHUNK)
    mesh = plsc.VectorSubcoreMesh(core_axis_name="c", subcore_axis_name="s")
    k = functools.partial(
        pl.kernel,
        mesh=mesh,
        out_type=jax.ShapeDtypeStruct((NUM_FIELDS, 8, NW, 8, BB), jnp.float32),
        scratch_types=[
            pltpu.VMEM((CHUNK,), jnp.int32),
            pltpu.VMEM((EMBED_DIM, CHUNK), jnp.int32),
            pltpu.VMEM((EMBED_DIM, CHUNK), jnp.int32),
            pltpu.VMEM((8, 8, CHUNK), jnp.float32),
            pltpu.VMEM((8, 8, CHUNK), jnp.float32),
            pltpu.SemaphoreType.DMA,
            pltpu.SemaphoreType.DMA,
            pltpu.SemaphoreType.DMA,
            pltpu.SemaphoreType.DMA,
        ],
        compiler_params=pltpu.CompilerParams(use_tc_tiling_on_sc=False),
    )(_body)
    out5 = k(x.T.reshape(-1), tflat, tdum)
    return (
        out5.transpose(0, 1, 3, 2, 4)
        .reshape(NUM_FIELDS, EMBED_DIM, B_ROWS)
        .transpose(2, 0, 1)
    )
